# baseline (speedup 1.0000x reference)
import functools
import operator

import jax
import jax.numpy as jnp
from jax.experimental import pallas as pl

_NQ = 8
_K = 1024
_D = 256
_BLK = 2048


def _stage_kernel(r_ref, rn_ref, c_ref, e_ref,
                  rnext_ref, qst_ref, idx_ref, loss_ref):
    r = r_ref[...]
    rn = rn_ref[...]
    c = c_ref[...]
    e = e_ref[...]
    s = jax.lax.dot_general(
        r, e, (((1,), (1,)), ((), ())),
        preferred_element_type=jnp.float32)
    dist = rn - 2.0 * s + c
    m = jnp.min(dist, axis=-1, keepdims=True)
    iota = jax.lax.broadcasted_iota(jnp.int32, (_BLK, _K), 1)
    idx = jnp.min(jnp.where(dist == m, iota, _K), axis=-1).astype(jnp.int32)
    oh = (iota == idx[:, None]).astype(jnp.float32)
    qv = jax.lax.dot_general(
        oh, e, (((1,), (0,)), ((), ())),
        preferred_element_type=jnp.float32,
        precision=jax.lax.Precision.HIGHEST)
    t = qv - r
    qst = r + t
    rnext_ref[...] = r - qst
    qst_ref[...] = qst
    idx_ref[:, 0] = idx
    loss_ref[0, 0, :] = jnp.sum(t * t)[None]


def _stage(r, rn, c, e, nb, tokens):
    return pl.pallas_call(
        _stage_kernel,
        grid=(nb,),
        in_specs=[
            pl.BlockSpec((_BLK, _D), lambda i: (i, 0)),
            pl.BlockSpec((_BLK, 1), lambda i: (i, 0)),
            pl.BlockSpec((1, _K), lambda i: (0, 0)),
            pl.BlockSpec((_K, _D), lambda i: (0, 0)),
        ],
        out_specs=[
            pl.BlockSpec((_BLK, _D), lambda i: (i, 0)),
            pl.BlockSpec((_BLK, _D), lambda i: (i, 0)),
            pl.BlockSpec((_BLK, 1), lambda i: (i, 0)),
            pl.BlockSpec((1, 1, 1), lambda i: (i, 0, 0)),
        ],
        out_shape=[
            jax.ShapeDtypeStruct((tokens, _D), jnp.float32),
            jax.ShapeDtypeStruct((tokens, _D), jnp.float32),
            jax.ShapeDtypeStruct((tokens, 1), jnp.int32),
            jax.ShapeDtypeStruct((nb, 1, 1), jnp.float32),
        ],
    )(r, rn, c, e)


def kernel(x, codebooks):
    b, n, d = x.shape
    tokens = b * n
    nb = tokens // _BLK
    r = x.reshape(tokens, d)
    qsts, idxs, losses = [], [], []
    for q in range(_NQ):
        rn = jnp.sum(r ** 2, axis=-1, keepdims=True)
        c = jnp.sum(codebooks[q] ** 2, axis=-1)[None, :]
        r, qst, idx, lossp = _stage(r, rn, c, codebooks[q], nb, tokens)
        qsts.append(qst)
        idxs.append(idx)
        losses.append(jnp.sum(lossp) / float(tokens * d))
    quantized_out = functools.reduce(operator.add, qsts).reshape(b, n, d)
    all_indices = jnp.concatenate(idxs, axis=-1).reshape(b, n, _NQ)
    all_losses = jnp.stack(losses, axis=-1)
    return quantized_out, all_indices, all_losses


# speedup vs baseline: 1.2896x; 1.2896x over previous
import jax
import jax.numpy as jnp
from jax.experimental import pallas as pl

_NQ = 8
_K = 1024
_D = 256
_BLK = 2048


def _stage_kernel(r_ref, rn_ref, c_ref, e_ref, qout_ref,
                  rnext_ref, qnew_ref, idx_ref, loss_ref):
    r = r_ref[...]
    rn = rn_ref[...]
    c = c_ref[...]
    e = e_ref[...]
    s = jax.lax.dot_general(
        r, e, (((1,), (1,)), ((), ())),
        preferred_element_type=jnp.float32)
    dist = rn - 2.0 * s + c
    m = jnp.min(dist, axis=-1, keepdims=True)
    iota = jax.lax.broadcasted_iota(jnp.int32, (_BLK, _K), 1)
    idx = jnp.min(jnp.where(dist == m, iota, _K), axis=-1).astype(jnp.int32)
    iota3 = jax.lax.broadcasted_iota(jnp.int32, (_BLK, 3 * _K), 1)
    oh3 = (jax.lax.rem(iota3, _K) == idx[:, None]).astype(jnp.bfloat16)
    e1 = e.astype(jnp.bfloat16)
    r1 = e - e1.astype(jnp.float32)
    e2 = r1.astype(jnp.bfloat16)
    e3 = (r1 - e2.astype(jnp.float32)).astype(jnp.bfloat16)
    ecat = jnp.concatenate([e1, e2, e3], axis=0)
    qv = jax.lax.dot_general(
        oh3, ecat, (((1,), (0,)), ((), ())),
        preferred_element_type=jnp.float32)
    t = qv - r
    qst = r + t
    rnext_ref[...] = r - qst
    qnew_ref[...] = qout_ref[...] + qst
    idx_ref[:, 0] = idx
    loss_ref[0, 0, :] = jnp.sum(t * t)[None]


def _stage(r, rn, c, e, qout, nb, tokens):
    return pl.pallas_call(
        _stage_kernel,
        grid=(nb,),
        in_specs=[
            pl.BlockSpec((_BLK, _D), lambda i: (i, 0)),
            pl.BlockSpec((_BLK, 1), lambda i: (i, 0)),
            pl.BlockSpec((1, _K), lambda i: (0, 0)),
            pl.BlockSpec((_K, _D), lambda i: (0, 0)),
            pl.BlockSpec((_BLK, _D), lambda i: (i, 0)),
        ],
        out_specs=[
            pl.BlockSpec((_BLK, _D), lambda i: (i, 0)),
            pl.BlockSpec((_BLK, _D), lambda i: (i, 0)),
            pl.BlockSpec((_BLK, 1), lambda i: (i, 0)),
            pl.BlockSpec((1, 1, 1), lambda i: (i, 0, 0)),
        ],
        out_shape=[
            jax.ShapeDtypeStruct((tokens, _D), jnp.float32),
            jax.ShapeDtypeStruct((tokens, _D), jnp.float32),
            jax.ShapeDtypeStruct((tokens, 1), jnp.int32),
            jax.ShapeDtypeStruct((nb, 1, 1), jnp.float32),
        ],
        input_output_aliases={4: 1},
    )(r, rn, c, e, qout)


def kernel(x, codebooks):
    b, n, d = x.shape
    tokens = b * n
    nb = tokens // _BLK
    r = x.reshape(tokens, d)
    qout = jnp.zeros((tokens, d), jnp.float32)
    idxs, losses = [], []
    for q in range(_NQ):
        rn = jnp.sum(r ** 2, axis=-1, keepdims=True)
        c = jnp.sum(codebooks[q] ** 2, axis=-1)[None, :]
        r, qout, idx, lossp = _stage(
            r, rn, c, codebooks[q], qout, nb, tokens)
        idxs.append(idx)
        losses.append(jnp.sum(lossp) / float(tokens * d))
    quantized_out = qout.reshape(b, n, d)
    all_indices = jnp.concatenate(idxs, axis=-1).reshape(b, n, _NQ)
    all_losses = jnp.stack(losses, axis=-1)
    return quantized_out, all_indices, all_losses


# speedup vs baseline: 1.4441x; 1.1198x over previous
import jax
import jax.numpy as jnp
from jax.experimental import pallas as pl

_NQ = 8
_K = 1024
_D = 256
_BLK = 2048


def _norms_t(a):
    t = (a * a).T
    acc = None
    for k in range(16):
        u = t[8 * k : 8 * k + 8, :] + t[128 + 8 * k : 136 + 8 * k, :]
        acc = u if acc is None else acc + u
    h = acc[0:4, :] + acc[4:8, :]
    h = h[0:2, :] + h[2:4, :]
    return h[0:1, :] + h[1:2, :]


def _rvq_kernel(x_ref, cb_ref, qout_ref, idx_ref, loss_ref):
    r = x_ref[...]
    qout = jnp.zeros_like(r)
    losses = []
    for q in range(_NQ):
        e = cb_ref[q]
        c = _norms_t(e)
        rn = _norms_t(r).T
        s = jax.lax.dot_general(
            r, e, (((1,), (1,)), ((), ())),
            preferred_element_type=jnp.float32)
        dist = rn - 2.0 * s + c
        m = jnp.min(dist, axis=-1, keepdims=True)
        iota = jax.lax.broadcasted_iota(jnp.int32, (_BLK, _K), 1)
        idx = jnp.min(jnp.where(dist == m, iota, _K), axis=-1).astype(jnp.int32)
        iota3 = jax.lax.broadcasted_iota(jnp.int32, (_BLK, 3 * _K), 1)
        oh3 = (jax.lax.rem(iota3, _K) == idx[:, None]).astype(jnp.bfloat16)
        e1 = e.astype(jnp.bfloat16)
        r1 = e - e1.astype(jnp.float32)
        e2 = r1.astype(jnp.bfloat16)
        e3 = (r1 - e2.astype(jnp.float32)).astype(jnp.bfloat16)
        ecat = jnp.concatenate([e1, e2, e3], axis=0)
        qv = jax.lax.dot_general(
            oh3, ecat, (((1,), (0,)), ((), ())),
            preferred_element_type=jnp.float32)
        t = qv - r
        qst = r + t
        losses.append(jnp.sum(t * t))
        qout = qout + qst
        r = r - qst
        idx_ref[:, q] = idx
    qout_ref[...] = qout
    loss_ref[0, 0, :] = jnp.stack(losses)


def kernel(x, codebooks):
    b, n, d = x.shape
    tokens = b * n
    nb = tokens // _BLK
    flat = x.reshape(tokens, d)
    qout, idx, lossp = pl.pallas_call(
        _rvq_kernel,
        grid=(nb,),
        in_specs=[
            pl.BlockSpec((_BLK, d), lambda i: (i, 0)),
            pl.BlockSpec((_NQ, _K, d), lambda i: (0, 0, 0)),
        ],
        out_specs=[
            pl.BlockSpec((_BLK, d), lambda i: (i, 0)),
            pl.BlockSpec((_BLK, _NQ), lambda i: (i, 0)),
            pl.BlockSpec((1, 1, _NQ), lambda i: (i, 0, 0)),
        ],
        out_shape=[
            jax.ShapeDtypeStruct((tokens, d), jnp.float32),
            jax.ShapeDtypeStruct((tokens, _NQ), jnp.int32),
            jax.ShapeDtypeStruct((nb, 1, _NQ), jnp.float32),
        ],
    )(flat, codebooks)
    quantized_out = qout.reshape(b, n, d)
    all_indices = idx.reshape(b, n, _NQ)
    all_losses = jnp.sum(lossp, axis=0)[0] / float(tokens * d)
    return quantized_out, all_indices, all_losses
